# baseline (speedup 1.0000x reference)
import functools

import jax
import jax.numpy as jnp
from jax.experimental import pallas as pl


_ITERS = 9


def _body(x_ref, o_ref, *, iters):
    x = x_ref[...]
    n = x.shape[1]
    rowsum = jnp.sum(x, axis=1, keepdims=True)
    xmax = jnp.max(x, axis=1, keepdims=True)
    alpha = 1.0 / jnp.abs(rowsum)
    z = x * alpha
    o_ref[...] = z
    zmax = xmax * alpha

    lo = zmax - 1.0
    hi = zmax
    sgn = jnp.sign(rowsum)
    t = jnp.maximum(lo, (sgn - 1.0) / n)
    tau = jnp.full_like(zmax, -3e38)
    for _ in range(iters):
        zz = o_ref[...]
        d = zz - t
        f = jnp.sum(jnp.maximum(d, 0.0), axis=1, keepdims=True)
        k = jnp.sum(jnp.where(d > 0.0, 1.0, 0.0), axis=1, keepdims=True)
        k = jnp.maximum(k, 1.0)
        t_n = t + (f - 1.0) / k
        ge = f >= 1.0
        lo = jnp.where(ge, t, lo)
        hi = jnp.where(ge, hi, t)
        tau = jnp.maximum(tau, t_n)
        t = jnp.maximum(t_n, 0.5 * (lo + hi))

    o_ref[...] = jnp.maximum(o_ref[...] - tau, 0.0)


def _kernel_tc(input):
    bs, dim = input.shape
    x = input.astype(jnp.float32)
    rows_per_block = 8
    grid = (bs // rows_per_block,)
    out = pl.pallas_call(
        functools.partial(_body, iters=_ITERS),
        grid=grid,
        in_specs=[pl.BlockSpec((rows_per_block, dim), lambda i: (i, 0))],
        out_specs=pl.BlockSpec((rows_per_block, dim), lambda i: (i, 0)),
        out_shape=jax.ShapeDtypeStruct((bs, dim), jnp.float32),
    )(x)
    return out


from jax import lax
from jax.experimental.pallas import tpu as pltpu
from jax.experimental.pallas import tpu_sc as plsc

_SC_NC = 2
_SC_NS = 16
_SC_L = 16


def _vdiv(a, b):
    av = jnp.broadcast_to(a, (_SC_L,))
    bv = jnp.broadcast_to(b, (_SC_L,))
    return av / bv


def _vec_sum(vec):
    a = vec[0]
    for i in range(1, _SC_L):
        a = a + vec[i]
    return a


def _vec_max(vec):
    a = vec[0]
    for i in range(1, _SC_L):
        a = jnp.maximum(a, vec[i])
    return a


def _sc_body(x_hbm, o_hbm, row_v, sem, *, bs, dim, iters):
    nc, ns, lanes = _SC_NC, _SC_NS, _SC_L
    nw = nc * ns
    wid = lax.axis_index("s") * nc + lax.axis_index("c")
    rows_per_w = bs // nw
    nchunk = dim // lanes

    for j in range(rows_per_w):
        r = wid * rows_per_w + j
        pltpu.async_copy(x_hbm.at[r], row_v, sem).wait()

        def sum_max_step(i, carry):
            acc_s, acc_m = carry
            v = row_v[pl.ds(i * lanes, lanes)]
            return acc_s + v, jnp.maximum(acc_m, v)

        acc_s0 = jnp.zeros((lanes,), jnp.float32)
        acc_m0 = jnp.full((lanes,), -3e38, jnp.float32)
        acc_s, acc_m = lax.fori_loop(0, nchunk, sum_max_step, (acc_s0, acc_m0))
        s = _vec_sum(acc_s)
        m = _vec_max(acc_m)
        c = jnp.abs(s)
        alpha_v = _vdiv(jnp.float32(1.0), c)

        lo_v = jnp.broadcast_to(
            jnp.maximum(m - c, (s - c) * jnp.float32(1.0 / dim)), (lanes,))
        hi_v = jnp.broadcast_to(m, (lanes,))
        u_v = lo_v
        tau_v = jnp.full((lanes,), -3e38, jnp.float32)
        for _ in range(iters):
            def pass_step(i, carry):
                accf, acck = carry
                v = row_v[pl.ds(i * lanes, lanes)]
                d = v - u_v
                accf = accf + jnp.maximum(d, 0.0)
                acck = acck + jnp.where(d > 0.0, 1.0, 0.0)
                return accf, acck

            accf, acck = lax.fori_loop(
                0, nchunk, pass_step,
                (jnp.zeros((lanes,), jnp.float32),
                 jnp.zeros((lanes,), jnp.float32)))
            g = _vec_sum(accf)
            k = jnp.maximum(_vec_sum(acck), 1.0)
            u_n_v = u_v + _vdiv(g - c, k)
            ge = g >= c
            lo_v = jnp.where(ge, u_v, lo_v)
            hi_v = jnp.where(ge, hi_v, u_v)
            tau_v = jnp.maximum(tau_v, u_n_v)
            u_v = jnp.maximum(u_n_v, 0.5 * (lo_v + hi_v))

        def out_step(i, _):
            v = row_v[pl.ds(i * lanes, lanes)]
            row_v[pl.ds(i * lanes, lanes)] = (
                jnp.maximum(v - tau_v, 0.0) * alpha_v)
            return 0

        lax.fori_loop(0, nchunk, out_step, 0)
        pltpu.async_copy(row_v, o_hbm.at[r], sem).wait()


def _kernel_sc(input):
    bs, dim = input.shape
    x = input.astype(jnp.float32)
    mesh = plsc.VectorSubcoreMesh(core_axis_name="c", subcore_axis_name="s")
    k = functools.partial(
        pl.kernel,
        mesh=mesh,
        out_type=jax.ShapeDtypeStruct((bs, dim), jnp.float32),
        scratch_types=[
            pltpu.VMEM((dim,), jnp.float32),
            pltpu.SemaphoreType.DMA,
        ],
    )(functools.partial(_sc_body, bs=bs, dim=dim, iters=_ITERS))
    return k(x)


kernel = _kernel_sc


# speedup vs baseline: 4.4947x; 4.4947x over previous
import functools

import jax
import jax.numpy as jnp
from jax.experimental import pallas as pl


_ITERS = 9


def _body(x_ref, o_ref, *, iters):
    x = x_ref[...]
    n = x.shape[1]
    rowsum = jnp.sum(x, axis=1, keepdims=True)
    xmax = jnp.max(x, axis=1, keepdims=True)
    alpha = 1.0 / jnp.abs(rowsum)
    z = x * alpha
    o_ref[...] = z
    zmax = xmax * alpha

    lo = zmax - 1.0
    hi = zmax
    sgn = jnp.sign(rowsum)
    t = jnp.maximum(lo, (sgn - 1.0) / n)
    tau = jnp.full_like(zmax, -3e38)
    for _ in range(iters):
        zz = o_ref[...]
        d = zz - t
        f = jnp.sum(jnp.maximum(d, 0.0), axis=1, keepdims=True)
        k = jnp.sum(jnp.where(d > 0.0, 1.0, 0.0), axis=1, keepdims=True)
        k = jnp.maximum(k, 1.0)
        t_n = t + (f - 1.0) / k
        ge = f >= 1.0
        lo = jnp.where(ge, t, lo)
        hi = jnp.where(ge, hi, t)
        tau = jnp.maximum(tau, t_n)
        t = jnp.maximum(t_n, 0.5 * (lo + hi))

    o_ref[...] = jnp.maximum(o_ref[...] - tau, 0.0)


def kernel(input):
    bs, dim = input.shape
    x = input.astype(jnp.float32)
    rows_per_block = 8
    grid = (bs // rows_per_block,)
    out = pl.pallas_call(
        functools.partial(_body, iters=_ITERS),
        grid=grid,
        in_specs=[pl.BlockSpec((rows_per_block, dim), lambda i: (i, 0))],
        out_specs=pl.BlockSpec((rows_per_block, dim), lambda i: (i, 0)),
        out_shape=jax.ShapeDtypeStruct((bs, dim), jnp.float32),
    )(x)
    return out


# speedup vs baseline: 4.8025x; 1.0685x over previous
import functools

import jax
import jax.numpy as jnp
from jax.experimental import pallas as pl


_ITERS = 8


def _body(x_ref, o_ref, *, iters):
    x = x_ref[...]
    n = x.shape[1]
    rowsum = jnp.sum(x, axis=1, keepdims=True)
    xmax = jnp.max(x, axis=1, keepdims=True)
    alpha = 1.0 / jnp.abs(rowsum)
    z = x * alpha
    o_ref[...] = z
    zmax = xmax * alpha

    lo = zmax - 1.0
    hi = zmax
    sgn = jnp.sign(rowsum)
    t = jnp.maximum(lo, (sgn - 1.0) / n)
    tau = jnp.full_like(zmax, -3e38)
    for _ in range(iters):
        zz = o_ref[...]
        d = zz - t
        f = jnp.sum(jnp.maximum(d, 0.0), axis=1, keepdims=True)
        k = jnp.sum(jnp.where(d > 0.0, 1.0, 0.0), axis=1, keepdims=True)
        k = jnp.maximum(k, 1.0)
        t_n = t + (f - 1.0) / k
        ge = f >= 1.0
        lo = jnp.where(ge, t, lo)
        hi = jnp.where(ge, hi, t)
        tau = jnp.maximum(tau, t_n)
        t = jnp.maximum(t_n, 0.5 * (lo + hi))

    o_ref[...] = jnp.maximum(o_ref[...] - tau, 0.0)


def kernel(input):
    bs, dim = input.shape
    x = input.astype(jnp.float32)
    rows_per_block = 8
    grid = (bs // rows_per_block,)
    out = pl.pallas_call(
        functools.partial(_body, iters=_ITERS),
        grid=grid,
        in_specs=[pl.BlockSpec((rows_per_block, dim), lambda i: (i, 0))],
        out_specs=pl.BlockSpec((rows_per_block, dim), lambda i: (i, 0)),
        out_shape=jax.ShapeDtypeStruct((bs, dim), jnp.float32),
    )(x)
    return out
